# baseline (speedup 1.0000x reference)
import jax
import jax.numpy as jnp
from jax import lax
from jax.experimental import pallas as pl
from jax.experimental.pallas import tpu as pltpu

V = 32000
B = 4096
LABEL_SMOOTHING = 0.1
CONFIDENCE = 1.0 - LABEL_SMOOTHING
IGNORE_COL = (-100) % V
SMOOTH = LABEL_SMOOTHING / (V - 2)

ROWS_PER_BLOCK = 64


def _loss_block_kernel(x_ref, t_ref, out_ref):
    i = pl.program_id(0)
    r = ROWS_PER_BLOCK
    x = x_ref[...]
    t = t_ref[0, 0, :]

    m = jnp.max(x, axis=1, keepdims=True)
    se = jnp.sum(jnp.exp(x - m), axis=1)
    lse = m[:, 0] + jnp.log(se)

    col = lax.broadcasted_iota(jnp.int32, (r, V), 1)
    w = jnp.where(col == t[:, None], CONFIDENCE, SMOOTH)
    fused = jnp.sum(x * w, axis=1)

    x_i = x[:, IGNORE_COL]
    is_i = (t == IGNORE_COL).astype(jnp.float32)

    slog_s = SMOOTH * jnp.log(SMOOTH)
    clog_c = CONFIDENCE * jnp.log(CONFIDENCE)
    base = (V - 2 + is_i) * slog_s + clog_c
    cross = fused - (1.0 - is_i) * SMOOTH * x_i - lse * (1.0 + is_i * SMOOTH)
    partial = jnp.sum(base - cross)

    @pl.when(i == 0)
    def _init():
        out_ref[0, 0] = 0.0

    out_ref[0, 0] += partial


@jax.jit
def kernel(output, target, one_hot):
    del one_hot
    b, v = output.shape
    r = ROWS_PER_BLOCK
    grid = b // r
    t3 = target.astype(jnp.int32).reshape(grid, 1, r)
    total = pl.pallas_call(
        _loss_block_kernel,
        grid=(grid,),
        in_specs=[
            pl.BlockSpec((r, v), lambda i: (i, 0)),
            pl.BlockSpec((1, 1, r), lambda i: (i, 0, 0)),
        ],
        out_specs=pl.BlockSpec(memory_space=pltpu.SMEM),
        out_shape=jax.ShapeDtypeStruct((1, 1), jnp.float32),
    )(output, t3)
    return (total[0, 0] / b).astype(jnp.float32)
